# baseline (speedup 1.0000x reference)
import jax
import jax.numpy as jnp
from jax.experimental import pallas as pl
from jax.experimental.pallas import tpu as pltpu

_TB = 4096
_P = jax.lax.Precision.DEFAULT


def _mlp_body(idx_ref, mt, pt, st, lt, w1, b1_, w2, b2_, w3, b3_, out_ref,
              tt, w2b, w3b):
    i = pl.program_id(0)

    @pl.when(i == 0)
    def _fold():
        z = jnp.zeros((6, 64), jnp.float32)
        for k, tref in enumerate((mt, pt, st, lt)):
            tab = jnp.concatenate([tref[...], z], axis=0)
            blk = jnp.dot(tab, w1[pl.ds(64 * k, 64), :],
                          preferred_element_type=jnp.float32,
                          precision=jax.lax.Precision.HIGHEST)
            if k == 0:
                row = jax.lax.broadcasted_iota(jnp.int32, (16, 1), 0)
                blk = blk + (row == 15).astype(jnp.float32) * b1_[...]
            tt[pl.ds(16 * k, 16), :] = blk.astype(jnp.bfloat16)
        w2b[...] = w2[...].astype(jnp.bfloat16)
        w3b[...] = w3[...].astype(jnp.bfloat16)

    idx = idx_ref[pl.ds(i * _TB, _TB), :].astype(jnp.float32)
    gk = jax.lax.broadcasted_iota(jnp.int32, (4, 64), 1) // 16
    e = (gk == jax.lax.broadcasted_iota(jnp.int32, (4, 64), 0))
    idxb = jnp.dot(idx, e.astype(jnp.float32),
                   preferred_element_type=jnp.float32, precision=_P)
    col = jax.lax.broadcasted_iota(jnp.int32, (_TB, 64), 1)
    o = (jnp.remainder(col, 16).astype(jnp.float32) == idxb) | (col == 15)
    onehot = o.astype(jnp.bfloat16)

    h = jnp.dot(onehot, tt[...], preferred_element_type=jnp.float32,
                precision=_P)
    h = jnp.maximum(h.astype(jnp.bfloat16), jnp.bfloat16(0.0))
    h = jnp.dot(h, w2b[...], preferred_element_type=jnp.float32,
                precision=_P) + b2_[...]
    h = jnp.maximum(h.astype(jnp.bfloat16), jnp.bfloat16(0.0))
    out_ref[...] = jnp.dot(h, w3b[...], preferred_element_type=jnp.float32,
                           precision=_P) + b3_[...]


def kernel(batch_tuta_feat, mag_table, prec_table, msd_table, lsd_table,
           W1, b1, W2, b2, W3, b3):
    B = batch_tuta_feat.shape[0]
    HID = W2.shape[0]
    OUTC = W3.shape[1]
    G = B // _TB

    b1r = b1.reshape(1, HID)
    b2r = b2.reshape(1, HID)
    b3r = b3.reshape(1, OUTC)

    full = lambda shape: pl.BlockSpec(shape, lambda i: (0, 0))
    return pl.pallas_call(
        _mlp_body,
        grid=(G,),
        in_specs=[
            full((B, 4)),
            full(mag_table.shape), full(prec_table.shape),
            full(msd_table.shape), full(lsd_table.shape),
            full(W1.shape),
            full((1, HID)),
            full(W2.shape),
            full((1, HID)),
            full(W3.shape),
            full((1, OUTC)),
        ],
        out_specs=pl.BlockSpec((_TB, OUTC), lambda i: (i, 0)),
        out_shape=jax.ShapeDtypeStruct((B, OUTC), jnp.float32),
        scratch_shapes=[pltpu.VMEM((64, HID), jnp.bfloat16),
                        pltpu.VMEM((HID, HID), jnp.bfloat16),
                        pltpu.VMEM((HID, OUTC), jnp.bfloat16)],
        compiler_params=pltpu.CompilerParams(
            dimension_semantics=("arbitrary",)),
    )(batch_tuta_feat, mag_table, prec_table, msd_table, lsd_table,
      W1, b1r, W2, b2r, W3, b3r)


# speedup vs baseline: 1.0538x; 1.0538x over previous
import jax
import jax.numpy as jnp
from jax.experimental import pallas as pl
from jax.experimental.pallas import tpu as pltpu

_TB = 4096
_P = jax.lax.Precision.DEFAULT


def _mlp_body(idx_ref, mt, pt, st, lt, w1, b1_, w2, b2_, w3, b3_, out_ref,
              tt, w2b, w3b):
    i = pl.program_id(0)

    @pl.when(i == 0)
    def _fold():
        z = jnp.zeros((6, 64), jnp.float32)
        for k, tref in enumerate((mt, pt, st, lt)):
            tab = jnp.concatenate([tref[...], z], axis=0)
            blk = jnp.dot(tab, w1[pl.ds(64 * k, 64), :],
                          preferred_element_type=jnp.float32,
                          precision=jax.lax.Precision.HIGHEST)
            if k == 0:
                row = jax.lax.broadcasted_iota(jnp.int32, (16, 1), 0)
                blk = blk + (row == 15).astype(jnp.float32) * b1_[...][None, :]
            tt[pl.ds(16 * k, 16), :] = blk.astype(jnp.bfloat16)
        w2b[...] = w2[...].astype(jnp.bfloat16)
        w3b[...] = w3[...].astype(jnp.bfloat16)

    idx = idx_ref[pl.ds(i * _TB, _TB), :].astype(jnp.float32)
    gk = jax.lax.broadcasted_iota(jnp.int32, (4, 64), 1) // 16
    e = (gk == jax.lax.broadcasted_iota(jnp.int32, (4, 64), 0))
    idxb = jnp.dot(idx, e.astype(jnp.float32),
                   preferred_element_type=jnp.float32, precision=_P)
    col = jax.lax.broadcasted_iota(jnp.int32, (_TB, 64), 1)
    o = (jnp.remainder(col, 16).astype(jnp.float32) == idxb) | (col == 15)
    onehot = o.astype(jnp.bfloat16)

    h = jnp.dot(onehot, tt[...], preferred_element_type=jnp.float32,
                precision=_P)
    h = jnp.maximum(h.astype(jnp.bfloat16), jnp.bfloat16(0.0))
    h = jnp.dot(h, w2b[...], preferred_element_type=jnp.float32,
                precision=_P) + b2_[...][None, :]
    h = jnp.maximum(h.astype(jnp.bfloat16), jnp.bfloat16(0.0))
    out_ref[...] = jnp.dot(h, w3b[...], preferred_element_type=jnp.float32,
                           precision=_P) + b3_[...][None, :]


def kernel(batch_tuta_feat, mag_table, prec_table, msd_table, lsd_table,
           W1, b1, W2, b2, W3, b3):
    B = batch_tuta_feat.shape[0]
    HID = W2.shape[0]
    OUTC = W3.shape[1]
    G = B // _TB

    full = lambda shape: pl.BlockSpec(shape, lambda i: (0, 0))
    full1 = lambda n: pl.BlockSpec((n,), lambda i: (0,))
    return pl.pallas_call(
        _mlp_body,
        grid=(G,),
        in_specs=[
            full((B, 4)),
            full(mag_table.shape), full(prec_table.shape),
            full(msd_table.shape), full(lsd_table.shape),
            full(W1.shape),
            full1(HID),
            full(W2.shape),
            full1(HID),
            full(W3.shape),
            full1(OUTC),
        ],
        out_specs=pl.BlockSpec((_TB, OUTC), lambda i: (i, 0)),
        out_shape=jax.ShapeDtypeStruct((B, OUTC), jnp.float32),
        scratch_shapes=[pltpu.VMEM((64, HID), jnp.bfloat16),
                        pltpu.VMEM((HID, HID), jnp.bfloat16),
                        pltpu.VMEM((HID, OUTC), jnp.bfloat16)],
        compiler_params=pltpu.CompilerParams(
            dimension_semantics=("arbitrary",)),
    )(batch_tuta_feat, mag_table, prec_table, msd_table, lsd_table,
      W1, b1, W2, b2, W3, b3)
